# baseline (speedup 1.0000x reference)
import functools

import jax
import jax.numpy as jnp
from jax import lax
from jax.experimental import pallas as pl
from jax.experimental.pallas import tpu as pltpu
from jax.experimental.pallas import tpu_sc as plsc

BATCH = 4096
SEQ = 200
D = 64
B_TOTAL = BATCH * SEQ
NC = 2
NS = 16
NW = NC * NS
B_PER_W = B_TOTAL // NW
CHUNK = 128
N_CHUNKS = B_PER_W // CHUNK


def _emb_body(idx_hbm, table_hbm, out_hbm, idx_v, rows_v, gsem):
    wid = lax.axis_index("s") * NC + lax.axis_index("c")
    base = wid * B_PER_W
    pltpu.sync_copy(idx_hbm.at[wid], idx_v)

    def chunk_body(j, carry):
        pltpu.async_copy(table_hbm.at[idx_v.at[j]], rows_v, gsem).wait()
        pltpu.sync_copy(rows_v, out_hbm.at[pl.ds(base + j * CHUNK, CHUNK)])
        return carry

    lax.fori_loop(0, N_CHUNKS, chunk_body, 0)


@jax.jit
def _emb_call(idx, table):
    mesh = plsc.VectorSubcoreMesh(core_axis_name="c", subcore_axis_name="s")
    return pl.kernel(
        _emb_body,
        mesh=mesh,
        out_type=jax.ShapeDtypeStruct((B_TOTAL, D), jnp.float32),
        scratch_types=[
            pltpu.VMEM((N_CHUNKS, CHUNK), jnp.int32),
            pltpu.VMEM((CHUNK, D), jnp.float32),
            pltpu.SemaphoreType.DMA,
        ],
        compiler_params=pltpu.CompilerParams(use_tc_tiling_on_sc=False),
    )(idx, table)


def kernel(x, table):
    idx = jnp.transpose(x).reshape(NW, N_CHUNKS, CHUNK)
    out = _emb_call(idx, table)
    return out.reshape(SEQ, BATCH, D)


# speedup vs baseline: 1.1172x; 1.1172x over previous
import jax
import jax.numpy as jnp
from jax import lax
from jax.experimental import pallas as pl
from jax.experimental.pallas import tpu as pltpu
from jax.experimental.pallas import tpu_sc as plsc

BATCH = 4096
SEQ = 200
D = 64
B_TOTAL = BATCH * SEQ
NC = 2
NS = 16
NW = NC * NS
B_PER_W = B_TOTAL // NW
CHUNK = 128
K = 2
SUPER = K * CHUNK
N_SUPER = B_PER_W // SUPER
N_CHUNKS = B_PER_W // CHUNK
NBUF = 4
AHEAD = 2
GROUPS = N_SUPER // NBUF


def _emb_body(idx_hbm, table_hbm, out_hbm, idx_v, rows_v, *sems):
    gsems, osems = sems[:NBUF], sems[NBUF:]
    wid = lax.axis_index("s") * NC + lax.axis_index("c")
    base = wid * B_PER_W
    pltpu.sync_copy(idx_hbm.at[wid], idx_v)

    def fire_gathers(s, b):
        for j in range(K):
            pltpu.async_copy(
                table_hbm.at[idx_v.at[s * K + j]],
                rows_v.at[pl.ds(b * SUPER + j * CHUNK, CHUNK)],
                gsems[b])

    def wait_gathers(b):
        pltpu.make_async_copy(
            table_hbm.at[pl.ds(0, SUPER)],
            rows_v.at[pl.ds(b * SUPER, SUPER)],
            gsems[b]).wait()

    def fire_scatter(s, b):
        pltpu.async_copy(
            rows_v.at[pl.ds(b * SUPER, SUPER)],
            out_hbm.at[pl.ds(base + s * SUPER, SUPER)],
            osems[b])

    def wait_scatter(b):
        pltpu.make_async_copy(
            rows_v.at[pl.ds(b * SUPER, SUPER)],
            out_hbm.at[pl.ds(0, SUPER)],
            osems[b]).wait()

    def process(s, b, wait_scat, fire_ahead):
        wait_gathers(b)
        fire_scatter(s, b)
        if fire_ahead:
            b2 = (b + AHEAD) % NBUF
            if wait_scat:
                wait_scatter(b2)
            fire_gathers(s + AHEAD, b2)

    for s in range(AHEAD):
        fire_gathers(s, s % NBUF)

    for b in range(NBUF):
        process(b, b, wait_scat=(b + AHEAD >= NBUF), fire_ahead=True)

    def group(m, carry):
        s0 = m * NBUF
        for b in range(NBUF):
            process(s0 + b, b, wait_scat=True, fire_ahead=True)
        return carry

    lax.fori_loop(1, GROUPS - 1, group, 0)

    s0 = (GROUPS - 1) * NBUF
    for b in range(NBUF):
        process(s0 + b, b, wait_scat=True,
                fire_ahead=(b + AHEAD < NBUF))
    for b in range(NBUF):
        wait_scatter(b)


@jax.jit
def _emb_call(idx, table):
    mesh = plsc.VectorSubcoreMesh(core_axis_name="c", subcore_axis_name="s")
    return pl.kernel(
        _emb_body,
        mesh=mesh,
        out_type=jax.ShapeDtypeStruct((B_TOTAL, D), jnp.float32),
        scratch_types=[
            pltpu.VMEM((N_CHUNKS, CHUNK), jnp.int32),
            pltpu.VMEM((NBUF * SUPER, D), jnp.float32),
        ] + [pltpu.SemaphoreType.DMA] * (2 * NBUF),
        compiler_params=pltpu.CompilerParams(use_tc_tiling_on_sc=False),
    )(idx, table)


def kernel(x, table):
    idx = jnp.transpose(x).reshape(NW, N_CHUNKS, CHUNK)
    out = _emb_call(idx, table)
    return out.reshape(SEQ, BATCH, D)


# speedup vs baseline: 1.3662x; 1.2229x over previous
import jax
import jax.numpy as jnp
from jax import lax
from jax.experimental import pallas as pl
from jax.experimental.pallas import tpu as pltpu
from jax.experimental.pallas import tpu_sc as plsc

BATCH = 4096
SEQ = 200
D = 64
DPAD = 128
NC = 2
NS = 16
NW = NC * NS
WB = BATCH // NW
NBUF = 4
AHEAD = 2
GROUPS = SEQ // NBUF


def _emb_body(xt_hbm, tbl_hbm, out_hbm, idx_v, rows_v, *sems):
    gsems, osems = sems[:NBUF], sems[NBUF:]
    wid = lax.axis_index("s") * NC + lax.axis_index("c")
    bbase = wid * WB
    pltpu.sync_copy(xt_hbm.at[:, pl.ds(bbase, WB)], idx_v)

    def fire_gather(s, slot):
        pltpu.async_copy(
            tbl_hbm.at[idx_v.at[s]],
            rows_v.at[pl.ds(slot * WB, WB)],
            gsems[slot])

    def wait_gather(slot):
        pltpu.make_async_copy(
            tbl_hbm.at[idx_v.at[0]],
            rows_v.at[pl.ds(slot * WB, WB)],
            gsems[slot]).wait()

    def fire_scatter(s, slot):
        pltpu.async_copy(
            rows_v.at[pl.ds(slot * WB, WB)],
            out_hbm.at[s, pl.ds(bbase, WB), :],
            osems[slot])

    def wait_scatter(slot):
        pltpu.make_async_copy(
            rows_v.at[pl.ds(slot * WB, WB)],
            out_hbm.at[0, pl.ds(bbase, WB), :],
            osems[slot]).wait()

    def process(s, slot, wait_scat, fire_ahead):
        wait_gather(slot)
        fire_scatter(s, slot)
        if fire_ahead:
            slot2 = (slot + AHEAD) % NBUF
            if wait_scat:
                wait_scatter(slot2)
            fire_gather(s + AHEAD, slot2)

    for s in range(AHEAD):
        fire_gather(s, s % NBUF)

    for b in range(NBUF):
        process(b, b, wait_scat=(b + AHEAD >= NBUF), fire_ahead=True)

    def group(m, carry):
        s0 = m * NBUF
        for b in range(NBUF):
            process(s0 + b, b, wait_scat=True, fire_ahead=True)
        return carry

    lax.fori_loop(1, GROUPS - 1, group, 0)

    s0 = (GROUPS - 1) * NBUF
    for b in range(NBUF):
        process(s0 + b, b, wait_scat=True, fire_ahead=(b + AHEAD < NBUF))
    for b in range(NBUF):
        wait_scatter(b)


@jax.jit
def _emb_call(xt, tbl):
    mesh = plsc.VectorSubcoreMesh(core_axis_name="c", subcore_axis_name="s")
    return pl.kernel(
        _emb_body,
        mesh=mesh,
        out_type=jax.ShapeDtypeStruct((SEQ, BATCH, DPAD), jnp.float32),
        scratch_types=[
            pltpu.VMEM((SEQ, WB), jnp.int32),
            pltpu.VMEM((NBUF * WB, DPAD), jnp.float32),
        ] + [pltpu.SemaphoreType.DMA] * (2 * NBUF),
        compiler_params=pltpu.CompilerParams(use_tc_tiling_on_sc=True),
    )(xt, tbl)


def kernel(x, table):
    xt = jnp.transpose(x)
    tbl = jnp.pad(table, ((0, 0), (0, DPAD - D)))
    return _emb_call(xt, tbl)[:, :, :D]
